# Initial kernel scaffold; baseline (speedup 1.0000x reference)
import jax
import jax.numpy as jnp
from jax.experimental import pallas as pl


def kernel(token_ids, weight):
    raise NotImplementedError("write your pallas kernel here")


# speedup vs baseline: 1.7964x; 1.7964x over previous
import functools

import jax
import jax.numpy as jnp
from jax import lax
from jax.experimental import pallas as pl
from jax.experimental.pallas import tpu as pltpu
from jax.experimental.pallas import tpu_sc as plsc

VOCAB = 1000000
D_MODEL = 64

NC = 2
NS = 16
NW = NC * NS

B_TOTAL = 16384 * 50
B_PER_W = B_TOTAL // NW

CHUNK = 512
GATHER = 128
N_GATHER = CHUNK // GATHER
N_CHUNKS = B_PER_W // CHUNK


def _body(idx_hbm, table_hbm, out_hbm, idx_v, rows_v, sem_idx, sem_g, sem_w):
    wid = lax.axis_index("s") * NC + lax.axis_index("c")
    base = wid * B_PER_W

    def step(j, carry):
        off = base + j * CHUNK
        pltpu.async_copy(idx_hbm.at[pl.ds(off, CHUNK)], idx_v, sem_idx).wait()
        handles = []
        for g in range(N_GATHER):
            s = pl.ds(g * GATHER, GATHER)
            handles.append(
                pltpu.async_copy(table_hbm.at[idx_v.at[s]], rows_v.at[s], sem_g)
            )
        for h in handles:
            h.wait()
        pltpu.async_copy(rows_v, out_hbm.at[pl.ds(off, CHUNK)], sem_w).wait()
        return carry

    lax.fori_loop(0, N_CHUNKS, step, 0, unroll=False)


@functools.partial(jax.jit, static_argnames=())
def kernel(token_ids, weight):
    idx_flat = token_ids.reshape(-1).astype(jnp.int32)
    mesh = plsc.VectorSubcoreMesh(core_axis_name="c", subcore_axis_name="s")
    out = pl.kernel(
        _body,
        out_type=jax.ShapeDtypeStruct((B_TOTAL, D_MODEL), jnp.float32),
        mesh=mesh,
        scratch_types=[
            pltpu.VMEM((CHUNK,), jnp.int32),
            pltpu.VMEM((CHUNK, D_MODEL), jnp.float32),
            pltpu.SemaphoreType.DMA,
            pltpu.SemaphoreType.DMA,
            pltpu.SemaphoreType.DMA,
        ],
        compiler_params=pltpu.CompilerParams(use_tc_tiling_on_sc=False),
    )(idx_flat, weight)
    return out.reshape(token_ids.shape + (D_MODEL,))


# speedup vs baseline: 1.8760x; 1.0443x over previous
import jax
import jax.numpy as jnp
from jax import lax
from jax.experimental import pallas as pl
from jax.experimental.pallas import tpu as pltpu
from jax.experimental.pallas import tpu_sc as plsc

D_MODEL = 64

NC = 2
NS = 16
NW = NC * NS

B_TOTAL = 16384 * 50
B_PER_W = B_TOTAL // NW

CHUNK = 512
GATHER = 128
N_GATHER = CHUNK // GATHER
N_CHUNKS = B_PER_W // CHUNK
N_PAIRS = N_CHUNKS // 2


def _body(idx_hbm, table_hbm, out_hbm,
          idx_all, rows0, rows1, sem_i, sem_g0, sem_g1, sem_w0, sem_w1):
    wid = lax.axis_index("s") * NC + lax.axis_index("c")
    base = wid * B_PER_W

    rows = (rows0, rows1)
    sem_g = (sem_g0, sem_g1)
    sem_w = (sem_w0, sem_w1)

    pltpu.async_copy(idx_hbm.at[pl.ds(base, B_PER_W)], idx_all, sem_i).wait()

    def fire_gathers(g, b):
        for k in range(N_GATHER):
            idx_s = idx_all.at[pl.ds(g * CHUNK + k * GATHER, GATHER)]
            pltpu.async_copy(table_hbm.at[idx_s], rows[b].at[pl.ds(k * GATHER, GATHER)], sem_g[b])

    def drain_gathers(b):
        for k in range(N_GATHER):
            idx_s = idx_all.at[pl.ds(k * GATHER, GATHER)]
            pltpu.make_async_copy(table_hbm.at[idx_s], rows[b].at[pl.ds(k * GATHER, GATHER)], sem_g[b]).wait()

    def fire_writeback(g, b):
        pltpu.async_copy(rows[b], out_hbm.at[pl.ds(base + g * CHUNK, CHUNK)], sem_w[b])

    def drain_writeback(b):
        pltpu.make_async_copy(rows[b], out_hbm.at[pl.ds(base, CHUNK)], sem_w[b]).wait()

    fire_gathers(0, 0)

    def step(t, carry):
        g0 = 2 * t
        @pl.when(t >= 1)
        def _():
            drain_writeback(1)
        fire_gathers(g0 + 1, 1)
        drain_gathers(0)
        fire_writeback(g0, 0)
        drain_writeback(0)
        @pl.when(t < N_PAIRS - 1)
        def _():
            fire_gathers(g0 + 2, 0)
        drain_gathers(1)
        fire_writeback(g0 + 1, 1)
        return carry

    lax.fori_loop(0, N_PAIRS, step, 0, unroll=False)
    drain_writeback(1)


def kernel(token_ids, weight):
    idx_flat = token_ids.reshape(-1).astype(jnp.int32)
    mesh = plsc.VectorSubcoreMesh(core_axis_name="c", subcore_axis_name="s")
    out = pl.kernel(
        _body,
        out_type=jax.ShapeDtypeStruct((B_TOTAL, D_MODEL), jnp.float32),
        mesh=mesh,
        scratch_types=[
            pltpu.VMEM((B_PER_W,), jnp.int32),
            pltpu.VMEM((CHUNK, D_MODEL), jnp.float32),
            pltpu.VMEM((CHUNK, D_MODEL), jnp.float32),
            pltpu.SemaphoreType.DMA,
            pltpu.SemaphoreType.DMA,
            pltpu.SemaphoreType.DMA,
            pltpu.SemaphoreType.DMA,
            pltpu.SemaphoreType.DMA,
        ],
        compiler_params=pltpu.CompilerParams(use_tc_tiling_on_sc=False),
    )(idx_flat, weight)
    return out.reshape(token_ids.shape + (D_MODEL,))
